# Initial kernel scaffold; baseline (speedup 1.0000x reference)
import jax
import jax.numpy as jnp
from jax.experimental import pallas as pl


def kernel(x, W1, b1, g1, be1, W2, b2, g2, be2, Wc, bc, edge_index):
    raise NotImplementedError("write your pallas kernel here")


# speedup vs baseline: 20.2453x; 20.2453x over previous
import functools

import jax
import jax.numpy as jnp
from jax import lax
from jax.experimental import pallas as pl
from jax.experimental.pallas import tpu as pltpu
from jax.experimental.pallas import tpu_sc as plsc

N = 10000
E = 320000
NPAD = 10240
NC = 2
NS = 16
CH = 128
NCH = 80
NBUF = 4
NG = NCH // NBUF
ET = NCH * CH
EPAD = NC * NS * ET
RPT = NPAD // NS
WD = 8
EPS = 1e-5
BR = 1024
GRID = NPAD // BR

_mesh = plsc.VectorSubcoreMesh(core_axis_name="c", subcore_axis_name="s")
_sc_params = pltpu.CompilerParams(use_tc_tiling_on_sc=False)


@functools.partial(
    pl.kernel,
    out_type=jax.ShapeDtypeStruct((NC, NPAD, WD), jnp.float32),
    mesh=_mesh,
    scratch_types=[
        pltpu.VMEM((NCH, CH), jnp.int32),
        pltpu.VMEM((CH, WD), jnp.float32),
        pltpu.VMEM_SHARED((NPAD, WD), jnp.float32),
        pltpu.SemaphoreType.DMA,
    ],
    compiler_params=_sc_params,
)
def _deg_kernel(ones_hbm, zeros_hbm, dst_hbm, out_hbm, dst_v, ones_v, deg_sh,
                sem):
    c = lax.axis_index("c")
    s = lax.axis_index("s")
    rbase = s * RPT
    pltpu.sync_copy(zeros_hbm.at[pl.ds(rbase, RPT)],
                    deg_sh.at[pl.ds(rbase, RPT)])
    pltpu.sync_copy(ones_hbm, ones_v)
    pltpu.sync_copy(dst_hbm.at[c, s], dst_v)
    plsc.subcore_barrier()

    def group(h, carry):
        for b in range(8):
            j = h * 8 + b
            pltpu.async_copy(ones_v, deg_sh.at[dst_v.at[j]], sem, add=True)
        for b in range(8):
            j = h * 8 + b
            pltpu.make_async_copy(ones_v, deg_sh.at[dst_v.at[j]], sem).wait()
        return carry

    lax.fori_loop(0, NCH // 8, group, 0)
    plsc.subcore_barrier()
    pltpu.sync_copy(deg_sh.at[pl.ds(rbase, RPT)],
                    out_hbm.at[c, pl.ds(rbase, RPT)])


def _make_scat(D):
    @functools.partial(
        pl.kernel,
        out_type=jax.ShapeDtypeStruct((NC, NPAD, D), jnp.float32),
        mesh=_mesh,
        scratch_types=[
            pltpu.VMEM((NCH, CH), jnp.int32),
            pltpu.VMEM((NCH, CH), jnp.int32),
            pltpu.VMEM((2 * NBUF, CH, D), jnp.float32),
            pltpu.VMEM_SHARED((NPAD, D), jnp.float32),
            pltpu.SemaphoreType.DMA,
            pltpu.SemaphoreType.DMA,
        ],
        compiler_params=_sc_params,
    )
    def scat(y_hbm, src_hbm, dst_hbm, out_hbm, src_v, dst_v, buf_v, acc_sh,
             sem0, sem1):
        c = lax.axis_index("c")
        s = lax.axis_index("s")
        rbase = s * RPT
        pltpu.sync_copy(y_hbm.at[pl.ds(rbase, RPT)],
                        acc_sh.at[pl.ds(rbase, RPT)])
        pltpu.sync_copy(src_hbm.at[c, s], src_v)
        pltpu.sync_copy(dst_hbm.at[c, s], dst_v)
        plsc.subcore_barrier()

        def fire(g, base, sem):
            for b in range(NBUF):
                pltpu.async_copy(y_hbm.at[src_v.at[g * NBUF + b]],
                                 buf_v.at[base + b], sem)

        def drain_and_scatter(g, base, sem):
            for b in range(NBUF):
                pltpu.make_async_copy(y_hbm.at[src_v.at[g * NBUF + b]],
                                      buf_v.at[base + b], sem).wait()
            for b in range(NBUF):
                pltpu.sync_copy(buf_v.at[base + b],
                                acc_sh.at[dst_v.at[g * NBUF + b]], add=True)

        fire(0, 0, sem0)

        def body(h, carry):
            g0 = 2 * h
            g1 = 2 * h + 1
            fire(g1, NBUF, sem1)
            drain_and_scatter(g0, 0, sem0)

            @pl.when(g1 + 1 < NG)
            def _():
                fire(g1 + 1, 0, sem0)

            drain_and_scatter(g1, NBUF, sem1)
            return carry

        lax.fori_loop(0, NG // 2, body, 0)
        plsc.subcore_barrier()
        pltpu.sync_copy(acc_sh.at[pl.ds(rbase, RPT)],
                        out_hbm.at[c, pl.ds(rbase, RPT)])

    return scat


_scat64 = _make_scat(64)
_scat32 = _make_scat(32)


def _tc_a_body(degp_ref, x_ref, w1_ref, dinv_ref, y1_ref):
    deg = degp_ref[0, :, 0:1] + degp_ref[1, :, 0:1] + 1.0
    dinv = lax.rsqrt(deg)
    dinv_ref[...] = dinv
    y1_ref[...] = jnp.dot(x_ref[...], w1_ref[...],
                          preferred_element_type=jnp.float32) * dinv


def _tc_mid_body(acc_ref, y_ref, dinv_ref, b_ref, g_ref, be_ref, w_ref,
                 out_ref):
    s = 1.0 / jnp.sqrt(1.0 + EPS)
    dinv = dinv_ref[...]
    v = (acc_ref[0] + acc_ref[1] - y_ref[...]) * dinv + b_ref[...]
    v = g_ref[...] * (v * s) + be_ref[...]
    v = jnp.maximum(v, 0.0)
    out_ref[...] = jnp.dot(v, w_ref[...],
                           preferred_element_type=jnp.float32) * dinv


def _tc_final_body(acc_ref, y_ref, dinv_ref, b_ref, g_ref, be_ref, wc_ref,
                   bc_ref, out_ref):
    s = 1.0 / jnp.sqrt(1.0 + EPS)
    dinv = dinv_ref[...]
    v = (acc_ref[0] + acc_ref[1] - y_ref[...]) * dinv + b_ref[...]
    v = g_ref[...] * (v * s) + be_ref[...]
    v = jnp.maximum(v, 0.0)
    out_ref[...] = jnp.dot(v, wc_ref[...],
                           preferred_element_type=jnp.float32) + bc_ref[...]


def _row_spec(d):
    return pl.BlockSpec((BR, d), lambda i: (i, 0))


def _full_spec(shape):
    return pl.BlockSpec(shape, lambda i: tuple(0 for _ in shape))


def _acc_spec(d):
    return pl.BlockSpec((NC, BR, d), lambda i: (0, i, 0))


def _tc_a(degp, xp, w1):
    return pl.pallas_call(
        _tc_a_body,
        grid=(GRID,),
        in_specs=[_acc_spec(WD), _row_spec(128), _full_spec((128, 64))],
        out_specs=[_row_spec(1), _row_spec(64)],
        out_shape=[
            jax.ShapeDtypeStruct((NPAD, 1), jnp.float32),
            jax.ShapeDtypeStruct((NPAD, 64), jnp.float32),
        ],
    )(degp, xp, w1)


def _tc_mid(acc, y1, dinv, b1, g1, be1, w2):
    return pl.pallas_call(
        _tc_mid_body,
        grid=(GRID,),
        in_specs=[_acc_spec(64), _row_spec(64), _row_spec(1),
                  _full_spec((1, 64)), _full_spec((1, 64)),
                  _full_spec((1, 64)), _full_spec((64, 32))],
        out_specs=_row_spec(32),
        out_shape=jax.ShapeDtypeStruct((NPAD, 32), jnp.float32),
    )(acc, y1, dinv, b1, g1, be1, w2)


def _tc_final(acc, y2, dinv, b2, g2, be2, wc, bc):
    return pl.pallas_call(
        _tc_final_body,
        grid=(GRID,),
        in_specs=[_acc_spec(32), _row_spec(32), _row_spec(1),
                  _full_spec((1, 32)), _full_spec((1, 32)),
                  _full_spec((1, 32)), _full_spec((32, 10)),
                  _full_spec((1, 10))],
        out_specs=_row_spec(10),
        out_shape=jax.ShapeDtypeStruct((NPAD, 10), jnp.float32),
    )(acc, y2, dinv, b2, g2, be2, wc, bc)


def kernel(x, W1, b1, g1, be1, W2, b2, g2, be2, Wc, bc, edge_index):
    src = edge_index[0]
    dst = edge_index[1]
    pad = EPAD - E
    srcp = jnp.concatenate([src, jnp.zeros((pad,), jnp.int32)])
    dstp = jnp.concatenate([dst, jnp.full((pad,), N, jnp.int32)])
    srcp = srcp.reshape(NC, NS, NCH, CH)
    dstp = dstp.reshape(NC, NS, NCH, CH)

    xp = jnp.zeros((NPAD, 128), jnp.float32).at[:N].set(x)
    ones = jnp.ones((CH, WD), jnp.float32)
    zeros = jnp.zeros((NPAD, WD), jnp.float32)

    degp = _deg_kernel(ones, zeros, dstp)
    dinv, y1 = _tc_a(degp, xp, W1)
    acc1 = _scat64(y1, srcp, dstp)
    y2 = _tc_mid(acc1, y1, dinv, b1.reshape(1, 64), g1.reshape(1, 64),
                 be1.reshape(1, 64), W2)
    acc2 = _scat32(y2, srcp, dstp)
    outp = _tc_final(acc2, y2, dinv, b2.reshape(1, 32), g2.reshape(1, 32),
                     be2.reshape(1, 32), Wc, bc.reshape(1, 10))
    return outp[:N]
